# baseline (speedup 1.0000x reference)
import functools

import jax
import jax.numpy as jnp
from jax import lax
from jax.experimental import pallas as pl
from jax.experimental.pallas import tpu as pltpu
from jax.experimental.pallas import tpu_sc as plsc

N = 10000
E = 320000
D = 128

NC = 2
NS = 16
NW = NC * NS
CH = 120
GRP = 6
NR = 3
NI = 6

NCHUNK = 84
EPW = NCHUNK * CH
EPAD = EPW * NW
NG = NCHUNK // GRP
TOTAL_CHUNKS = EPAD // CH

NPAD = 10368
RPT = NPAD // NS


def _spmm_sc(table, epack, zeros):
    mesh = plsc.VectorSubcoreMesh(core_axis_name="c", subcore_axis_name="s")

    @functools.partial(
        pl.kernel,
        out_type=jax.ShapeDtypeStruct((NC * NPAD, D), jnp.float32),
        mesh=mesh,
        scratch_types=(
            [pltpu.VMEM((2, CH), jnp.int32) for _ in range(NI)]
            + [pltpu.VMEM((CH, D), jnp.float32) for _ in range(NR)]
            + [pltpu.VMEM_SHARED((NPAD, D), jnp.float32)]
            + [pltpu.SemaphoreType.DMA for _ in range(NI + 2 * NR)]
        ),
    )
    def spmm(table_hbm, epack_hbm, zeros_hbm, out_hbm, *refs):
        idxb = refs[0:NI]
        rows = refs[NI:NI + NR]
        acc = refs[NI + NR]
        isem = refs[NI + NR + 1:2 * NI + NR + 1]
        gsem = refs[2 * NI + NR + 1:2 * NI + 2 * NR + 1]
        ssem = refs[2 * NI + 2 * NR + 1:2 * NI + 3 * NR + 1]

        cid = lax.axis_index("c")
        sid = lax.axis_index("s")
        wid = sid * NC + cid
        cbase = wid * NCHUNK

        def idx_start(c, s):
            pltpu.async_copy(epack_hbm.at[c], idxb[s], isem[s])

        def idx_wait(c, s):
            pltpu.make_async_copy(epack_hbm.at[c], idxb[s], isem[s]).wait()

        def gather_start(s, p):
            pltpu.async_copy(table_hbm.at[idxb[s].at[0]], rows[p], gsem[p])

        def gather_wait(s, p):
            pltpu.make_async_copy(table_hbm.at[idxb[s].at[0]], rows[p],
                                  gsem[p]).wait()

        def scatter_start(s, p):
            pltpu.async_copy(rows[p], acc.at[idxb[s].at[1]], ssem[p],
                             add=True)

        def scatter_wait(s, p):
            pltpu.make_async_copy(rows[p], acc.at[idxb[s].at[1]],
                                  ssem[p]).wait()

        def step(j, pos, wait_s2, idx_pre, gather_next):
            p = pos % NR
            pn = (pos + 1) % NR
            s_cur = pos % NI
            s_next = (pos + 1) % NI
            s_pre = (pos + 4) % NI
            if wait_s2:
                scatter_wait(s_pre, pn)
            if idx_pre:
                idx_start(j + 4, s_pre)
            if gather_next:
                idx_wait(j + 1, s_next)
                gather_start(s_next, pn)
            gather_wait(s_cur, p)
            scatter_start(s_cur, p)

        for s in range(4):
            idx_start(cbase + s, s)
        idx_wait(cbase + 0, 0)
        gather_start(0, 0)

        pltpu.sync_copy(zeros_hbm.at[pl.ds(sid * RPT, RPT)],
                        acc.at[pl.ds(sid * RPT, RPT)])
        plsc.subcore_barrier()

        step(cbase + 0, 0, False, True, True)
        step(cbase + 1, 1, False, True, True)
        for pos in range(2, GRP):
            step(cbase + pos, pos, True, True, True)

        def body(t, carry):
            j0 = cbase + t * GRP
            for pos in range(GRP):
                step(j0 + pos, pos, True, True, True)
            return carry

        lax.fori_loop(1, NG - 1, body, 0)

        j0 = cbase + (NG - 1) * GRP
        step(j0 + 0, 0, True, True, True)
        step(j0 + 1, 1, True, True, True)
        step(j0 + 2, 2, True, False, True)
        step(j0 + 3, 3, True, False, True)
        step(j0 + 4, 4, True, False, True)
        step(j0 + 5, 5, True, False, False)
        scatter_wait((GRP - 2) % NI, (GRP - 2) % NR)
        scatter_wait((GRP - 1) % NI, (GRP - 1) % NR)

        plsc.subcore_barrier()

        pltpu.sync_copy(acc.at[pl.ds(sid * RPT, RPT)],
                        out_hbm.at[pl.ds(cid * NPAD + sid * RPT, RPT)])

    return spmm(table, epack, zeros)


def _dense_kernel(p0_ref, p1_ref, h_ref, wrel_ref, wroot_ref, b_ref, o_ref,
                  *, relu):
    agg = p0_ref[...] + p1_ref[...]
    y = lax.dot_general(agg, wrel_ref[...], (((1,), (1,)), ((), ())),
                        preferred_element_type=jnp.float32)
    y += lax.dot_general(h_ref[...], wroot_ref[...], (((1,), (1,)), ((), ())),
                         preferred_element_type=jnp.float32)
    y += b_ref[...]
    o_ref[...] = jnp.maximum(y, 0.0) if relu else y


def _dense_tc(parts, h, w_rel, w_root, b, relu):
    blk = RPT
    grid = NPAD // blk
    row_spec = pl.BlockSpec((blk, D), lambda i: (i, 0))
    full_spec = pl.BlockSpec((D, D), lambda i: (0, 0))
    return pl.pallas_call(
        functools.partial(_dense_kernel, relu=relu),
        grid=(grid,),
        in_specs=[row_spec,
                  pl.BlockSpec((blk, D), lambda i: (i + grid, 0)),
                  row_spec, full_spec, full_spec,
                  pl.BlockSpec((1, D), lambda i: (0, 0))],
        out_specs=row_spec,
        out_shape=jax.ShapeDtypeStruct((NPAD, D), jnp.float32),
    )(parts, parts, h, w_rel, w_root, b)


def kernel(x, edge_index, W_rel1, b_rel1, W_root1, W_rel2, b_rel2, W_root2):
    src = edge_index[0].astype(jnp.int32)
    dst = edge_index[1].astype(jnp.int32)

    npad_e = EPAD - E
    pad_src = (jnp.arange(npad_e, dtype=jnp.int32) * 37) % N
    pad_dst = N + (jnp.arange(npad_e, dtype=jnp.int32) % (NPAD - N))
    src_p = jnp.concatenate([src, pad_src]).reshape(TOTAL_CHUNKS, 1, CH)
    dst_p = jnp.concatenate([dst, pad_dst]).reshape(TOTAL_CHUNKS, 1, CH)
    epack = jnp.concatenate([src_p, dst_p], axis=1)

    zeros = jnp.zeros((NPAD, D), jnp.float32)
    b1 = b_rel1.reshape(1, D)
    b2 = b_rel2.reshape(1, D)
    xp = jnp.pad(x, ((0, NPAD - N), (0, 0)))

    parts = _spmm_sc(xp, epack, zeros)
    h = _dense_tc(parts, xp, W_rel1, W_root1, b1, relu=True)
    parts2 = _spmm_sc(h, epack, zeros)
    out = _dense_tc(parts2, h, W_rel2, W_root2, b2, relu=False)
    return out[:N]


# speedup vs baseline: 1.0341x; 1.0341x over previous
import functools

import jax
import jax.numpy as jnp
from jax import lax
from jax.experimental import pallas as pl
from jax.experimental.pallas import tpu as pltpu
from jax.experimental.pallas import tpu_sc as plsc

N = 10000
E = 320000
D = 128

NC = 2
NS = 16
NW = NC * NS
CH = 120
GRP = 6
NR = 3
NI = 6

NCHUNK = 84
EPW = NCHUNK * CH
EPAD = EPW * NW
NG = NCHUNK // GRP
TOTAL_CHUNKS = EPAD // CH

NPAD = 10368
RPT = NPAD // NS


def _spmm_sc(table, epack):
    mesh = plsc.VectorSubcoreMesh(core_axis_name="c", subcore_axis_name="s")

    @functools.partial(
        pl.kernel,
        out_type=jax.ShapeDtypeStruct((NC * NPAD, D), jnp.float32),
        mesh=mesh,
        scratch_types=(
            [pltpu.VMEM((2, CH), jnp.int32) for _ in range(NI)]
            + [pltpu.VMEM((CH, D), jnp.float32) for _ in range(NR)]
            + [pltpu.VMEM_SHARED((NPAD, D), jnp.float32)]
            + [pltpu.SemaphoreType.DMA
               for _ in range(NI + 2 * NR + 1)]
        ),
    )
    def spmm(table_hbm, epack_hbm, out_hbm, *refs):
        idxb = refs[0:NI]
        rows = refs[NI:NI + NR]
        acc = refs[NI + NR]
        isem = refs[NI + NR + 1:2 * NI + NR + 1]
        gsem = refs[2 * NI + NR + 1:2 * NI + 2 * NR + 1]
        ssem = refs[2 * NI + 2 * NR + 1:2 * NI + 3 * NR + 1]
        zsem = refs[2 * NI + 3 * NR + 1]

        cid = lax.axis_index("c")
        sid = lax.axis_index("s")
        wid = sid * NC + cid
        cbase = wid * NCHUNK

        def idx_start(c, s):
            pltpu.async_copy(epack_hbm.at[c], idxb[s], isem[s])

        def idx_wait(c, s):
            pltpu.make_async_copy(epack_hbm.at[c], idxb[s], isem[s]).wait()

        def gather_start(s, p):
            pltpu.async_copy(table_hbm.at[idxb[s].at[0]], rows[p], gsem[p])

        def gather_wait(s, p):
            pltpu.make_async_copy(table_hbm.at[idxb[s].at[0]], rows[p],
                                  gsem[p]).wait()

        def scatter_start(s, p):
            pltpu.async_copy(rows[p], acc.at[idxb[s].at[1]], ssem[p],
                             add=True)

        def scatter_wait(s, p):
            pltpu.make_async_copy(rows[p], acc.at[idxb[s].at[1]],
                                  ssem[p]).wait()

        def step(j, pos, wait_s2, idx_pre, gather_next):
            p = pos % NR
            pn = (pos + 1) % NR
            s_cur = pos % NI
            s_next = (pos + 1) % NI
            s_pre = (pos + 4) % NI
            if wait_s2:
                scatter_wait(s_pre, pn)
            if idx_pre:
                idx_start(j + 4, s_pre)
            if gather_next:
                idx_wait(j + 1, s_next)
                gather_start(s_next, pn)
            gather_wait(s_cur, p)
            scatter_start(s_cur, p)

        for s in range(4):
            idx_start(cbase + s, s)
        idx_wait(cbase + 0, 0)
        gather_start(0, 0)

        zrow = rows[NR - 1]

        def zbody(i, carry):
            for k in range(D // 16):
                zrow[i, pl.ds(16 * k, 16)] = jnp.zeros((16,), jnp.float32)
            return carry

        lax.fori_loop(0, CH, zbody, 0)
        nfull = RPT // CH
        for z in range(nfull):
            pltpu.async_copy(zrow, acc.at[pl.ds(sid * RPT + z * CH, CH)],
                             zsem)
        rem = RPT - nfull * CH
        pltpu.async_copy(zrow.at[pl.ds(0, rem)],
                         acc.at[pl.ds(sid * RPT + nfull * CH, rem)], zsem)
        for z in range(nfull):
            pltpu.make_async_copy(
                zrow, acc.at[pl.ds(sid * RPT + z * CH, CH)], zsem).wait()
        pltpu.make_async_copy(
            zrow.at[pl.ds(0, rem)],
            acc.at[pl.ds(sid * RPT + nfull * CH, rem)], zsem).wait()
        plsc.subcore_barrier()

        step(cbase + 0, 0, False, True, True)
        step(cbase + 1, 1, False, True, True)
        for pos in range(2, GRP):
            step(cbase + pos, pos, True, True, True)

        def body(t, carry):
            j0 = cbase + t * GRP
            for pos in range(GRP):
                step(j0 + pos, pos, True, True, True)
            return carry

        lax.fori_loop(1, NG - 1, body, 0)

        j0 = cbase + (NG - 1) * GRP
        step(j0 + 0, 0, True, True, True)
        step(j0 + 1, 1, True, True, True)
        step(j0 + 2, 2, True, False, True)
        step(j0 + 3, 3, True, False, True)
        step(j0 + 4, 4, True, False, True)
        step(j0 + 5, 5, True, False, False)
        scatter_wait((GRP - 2) % NI, (GRP - 2) % NR)
        scatter_wait((GRP - 1) % NI, (GRP - 1) % NR)

        plsc.subcore_barrier()

        pltpu.sync_copy(acc.at[pl.ds(sid * RPT, RPT)],
                        out_hbm.at[pl.ds(cid * NPAD + sid * RPT, RPT)])

    return spmm(table, epack)


def _dense_kernel(p0_ref, p1_ref, h_ref, wrel_ref, wroot_ref, b_ref, o_ref,
                  *, relu):
    agg = p0_ref[...] + p1_ref[...]
    y = lax.dot_general(agg, wrel_ref[...], (((1,), (1,)), ((), ())),
                        preferred_element_type=jnp.float32)
    y += lax.dot_general(h_ref[...], wroot_ref[...], (((1,), (1,)), ((), ())),
                         preferred_element_type=jnp.float32)
    y += b_ref[...]
    o_ref[...] = jnp.maximum(y, 0.0) if relu else y


def _dense_tc(parts, h, w_rel, w_root, b, relu):
    blk = RPT
    grid = NPAD // blk
    row_spec = pl.BlockSpec((blk, D), lambda i: (i, 0))
    full_spec = pl.BlockSpec((D, D), lambda i: (0, 0))
    return pl.pallas_call(
        functools.partial(_dense_kernel, relu=relu),
        grid=(grid,),
        in_specs=[row_spec,
                  pl.BlockSpec((blk, D), lambda i: (i + grid, 0)),
                  row_spec, full_spec, full_spec,
                  pl.BlockSpec((1, D), lambda i: (0, 0))],
        out_specs=row_spec,
        out_shape=jax.ShapeDtypeStruct((NPAD, D), jnp.float32),
    )(parts, parts, h, w_rel, w_root, b)


def kernel(x, edge_index, W_rel1, b_rel1, W_root1, W_rel2, b_rel2, W_root2):
    src = edge_index[0].astype(jnp.int32)
    dst = edge_index[1].astype(jnp.int32)

    npad_e = EPAD - E
    pad_src = (jnp.arange(npad_e, dtype=jnp.int32) * 37) % N
    pad_dst = N + (jnp.arange(npad_e, dtype=jnp.int32) % (NPAD - N))
    src_p = jnp.concatenate([src, pad_src]).reshape(TOTAL_CHUNKS, 1, CH)
    dst_p = jnp.concatenate([dst, pad_dst]).reshape(TOTAL_CHUNKS, 1, CH)
    epack = jnp.concatenate([src_p, dst_p], axis=1)

    b1 = b_rel1.reshape(1, D)
    b2 = b_rel2.reshape(1, D)
    xp = jnp.pad(x, ((0, NPAD - N), (0, 0)))

    parts = _spmm_sc(xp, epack)
    h = _dense_tc(parts, xp, W_rel1, W_root1, b1, relu=True)
    parts2 = _spmm_sc(h, epack)
    out = _dense_tc(parts2, h, W_rel2, W_root2, b2, relu=False)
    return out[:N]
